# baseline (speedup 1.0000x reference)
import functools

import jax
import jax.numpy as jnp
from jax import lax
from jax.experimental import pallas as pl
from jax.experimental.pallas import tpu as pltpu
from jax.experimental.pallas import tpu_sc as plsc

N = 10000
NP = 10240
E = 320000
D1 = 128
NC = 2
NS = 16
NW = NC * NS
EPW = E // NW
CH = 80
NCHUNK = EPW // CH
RPT = NP // NS
RB = 1000


CPS = RPT // CH
VPC = CH // 16


def _zero_stripe(accum, z128_h, sid, sem):
  ds = []
  for j in range(CPS):
    ds.append(pltpu.async_copy(
        z128_h, accum.at[pl.ds(sid * RPT + j * CH, CH)], sem))
  for d in ds:
    d.wait()


def _copy_out(accum, out_hbm, cid, sid, sem):
  ds = []
  for j in range(CPS):
    base = sid * RPT + j * CH
    ds.append(pltpu.async_copy(
        accum.at[pl.ds(base, CH)], out_hbm.at[cid, pl.ds(base, CH)], sem))
  for d in ds:
    d.wait()


def _prop_pass(table_hbm, row_hbm, col_hbm, accum, slots, wid):
  def idx_start(k, s):
    cb, rb, _, _, isem = s
    base = wid * EPW + k * CH
    r = pltpu.async_copy(row_hbm.at[pl.ds(base, CH)], rb, isem)
    c = pltpu.async_copy(col_hbm.at[pl.ds(base, CH)], cb, isem)
    return r, c

  def idx_wait(d):
    d[0].wait()
    d[1].wait()

  def gather_start(s):
    cb, _, gb, gsem, _ = s
    return pltpu.async_copy(table_hbm.at[cb], gb, gsem)

  def scatter(s):
    _, rb, gb, _, _ = s
    pltpu.sync_copy(gb, accum.at[rb], add=True)

  def gather_wait(s):
    pltpu.make_async_copy(table_hbm.at[s[0]], s[2], s[3]).wait()

  s0, s1, s2 = slots
  d0 = idx_start(0, s0)
  d1 = idx_start(1, s1)
  d2 = idx_start(2, s2)
  idx_wait(d0)
  gather_start(s0)
  idx_wait(d1)
  gather_start(s1)

  def body(i, c):
    k = 3 * i
    gather_wait(s0)
    scatter(s0)
    idx_start(k + 3, s0)
    idx_wait(d2)
    gather_start(s2)
    gather_wait(s1)
    scatter(s1)
    idx_start(k + 4, s1)
    idx_wait(d0)
    gather_start(s0)
    gather_wait(s2)
    scatter(s2)
    idx_start(k + 5, s2)
    idx_wait(d1)
    gather_start(s1)
    return c

  lax.fori_loop(0, (NCHUNK - 5) // 3, body, 0)

  gather_wait(s0)
  scatter(s0)
  d3 = idx_start(NCHUNK - 2, s0)
  idx_wait(d2)
  gather_start(s2)
  gather_wait(s1)
  scatter(s1)
  d4 = idx_start(NCHUNK - 1, s1)
  idx_wait(d3)
  gather_start(s0)
  gather_wait(s2)
  scatter(s2)
  idx_wait(d4)
  gather_start(s1)
  gather_wait(s0)
  scatter(s0)
  gather_wait(s1)
  scatter(s1)


def _deg_pass(row_hbm, accum, ones_b, rb0, rb1, isem0, isem1, wid):
  def load(k, rb, sem):
    base = wid * EPW + k * CH
    return pltpu.async_copy(row_hbm.at[pl.ds(base, CH)], rb, sem)

  pltpu.sync_copy(row_hbm.at[pl.ds(wid * EPW, CH)], rb0)

  def body(i, c):
    k0 = 2 * i
    l1 = load(k0 + 1, rb1, isem1)
    pltpu.sync_copy(ones_b, accum.at[rb0], add=True)
    l1.wait()
    l0 = load(k0 + 2, rb0, isem0)
    pltpu.sync_copy(ones_b, accum.at[rb1], add=True)
    l0.wait()
    return c

  lax.fori_loop(0, (NCHUNK - 1) // 2, body, 0)
  pltpu.sync_copy(ones_b, accum.at[rb0], add=True)


def _sc1_body(xwl, xwh, row_h, col_h, z128_h, ones_h,
              low_o, high_o, deg_o,
              cb0, rb0, gb0, cb1, rb1, gb1, cb2, rb2, gb2, accum,
              gsem0, gsem1, gsem2, isem0, isem1, isem2):
  cid = lax.axis_index("c")
  sid = lax.axis_index("s")
  wid = sid * NC + cid
  slots = ((cb0, rb0, gb0, gsem0, isem0),
           (cb1, rb1, gb1, gsem1, isem1),
           (cb2, rb2, gb2, gsem2, isem2))

  _zero_stripe(accum, z128_h, sid, gsem0)
  pltpu.sync_copy(ones_h, gb0)
  plsc.subcore_barrier()

  _deg_pass(row_h, accum, gb0, rb0, rb1, isem0, isem1, wid)
  plsc.subcore_barrier()
  _copy_out(accum, deg_o, cid, sid, gsem0)
  _zero_stripe(accum, z128_h, sid, gsem0)
  plsc.subcore_barrier()

  _prop_pass(xwl, row_h, col_h, accum, slots, wid)
  plsc.subcore_barrier()
  _copy_out(accum, low_o, cid, sid, gsem0)
  _zero_stripe(accum, z128_h, sid, gsem0)
  plsc.subcore_barrier()

  _prop_pass(xwh, row_h, col_h, accum, slots, wid)
  plsc.subcore_barrier()
  _copy_out(accum, high_o, cid, sid, gsem0)


def _sc2_body(hlh, row_h, col_h, z128_h, out_o,
              cb0, rb0, gb0, cb1, rb1, gb1, cb2, rb2, gb2, accum,
              gsem0, gsem1, gsem2, isem0, isem1, isem2):
  cid = lax.axis_index("c")
  sid = lax.axis_index("s")
  wid = sid * NC + cid
  slots = ((cb0, rb0, gb0, gsem0, isem0),
           (cb1, rb1, gb1, gsem1, isem1),
           (cb2, rb2, gb2, gsem2, isem2))

  _zero_stripe(accum, z128_h, sid, gsem0)
  plsc.subcore_barrier()

  _prop_pass(hlh, row_h, col_h, accum, slots, wid)
  plsc.subcore_barrier()
  _copy_out(accum, out_o, cid, sid, gsem0)


@functools.cache
def _sc_kernels():
  mesh = plsc.VectorSubcoreMesh(core_axis_name="c", subcore_axis_name="s",
                                num_cores=NC, num_subcores=NS)
  sc1 = functools.partial(
      pl.kernel,
      out_type=[
          jax.ShapeDtypeStruct((NC, NP, D1), jnp.float32),
          jax.ShapeDtypeStruct((NC, NP, D1), jnp.float32),
          jax.ShapeDtypeStruct((NC, NP, D1), jnp.float32),
      ],
      mesh=mesh,
      scratch_types=[
          pltpu.VMEM((CH,), jnp.int32),
          pltpu.VMEM((CH,), jnp.int32),
          pltpu.VMEM((CH, D1), jnp.float32),
          pltpu.VMEM((CH,), jnp.int32),
          pltpu.VMEM((CH,), jnp.int32),
          pltpu.VMEM((CH, D1), jnp.float32),
          pltpu.VMEM((CH,), jnp.int32),
          pltpu.VMEM((CH,), jnp.int32),
          pltpu.VMEM((CH, D1), jnp.float32),
          pltpu.VMEM_SHARED((NP, D1), jnp.float32),
          pltpu.SemaphoreType.DMA,
          pltpu.SemaphoreType.DMA,
          pltpu.SemaphoreType.DMA,
          pltpu.SemaphoreType.DMA,
          pltpu.SemaphoreType.DMA,
          pltpu.SemaphoreType.DMA,
      ],
  )(_sc1_body)

  sc2 = functools.partial(
      pl.kernel,
      out_type=[jax.ShapeDtypeStruct((NC, NP, D1), jnp.float32)],
      mesh=mesh,
      scratch_types=[
          pltpu.VMEM((CH,), jnp.int32),
          pltpu.VMEM((CH,), jnp.int32),
          pltpu.VMEM((CH, D1), jnp.float32),
          pltpu.VMEM((CH,), jnp.int32),
          pltpu.VMEM((CH,), jnp.int32),
          pltpu.VMEM((CH, D1), jnp.float32),
          pltpu.VMEM((CH,), jnp.int32),
          pltpu.VMEM((CH,), jnp.int32),
          pltpu.VMEM((CH, D1), jnp.float32),
          pltpu.VMEM_SHARED((NP, D1), jnp.float32),
          pltpu.SemaphoreType.DMA,
          pltpu.SemaphoreType.DMA,
          pltpu.SemaphoreType.DMA,
          pltpu.SemaphoreType.DMA,
          pltpu.SemaphoreType.DMA,
          pltpu.SemaphoreType.DMA,
      ],
  )(_sc2_body)
  return sc1, sc2


def _pre_body(x_ref, wl_ref, wh_ref, wm_ref, ol_ref, oh_ref, om_ref):
  xb = x_ref[...]
  ol_ref[...] = jnp.dot(xb, wl_ref[...], preferred_element_type=jnp.float32)
  oh_ref[...] = jnp.dot(xb, wh_ref[...], preferred_element_type=jnp.float32)
  om_ref[...] = jnp.dot(xb, wm_ref[...], preferred_element_type=jnp.float32)


def _mix(low, high, mlp, al, ah, am, av_ref):
  sl = jnp.sum(low * al, axis=1, keepdims=True)
  sh = jnp.sum(high * ah, axis=1, keepdims=True)
  sm = jnp.sum(mlp * am, axis=1, keepdims=True)
  g0 = jax.nn.sigmoid(sl)
  g1 = jax.nn.sigmoid(sh)
  g2 = jax.nn.sigmoid(sm)
  t = [(g0 * av_ref[0, j] + g1 * av_ref[1, j] + g2 * av_ref[2, j]) / 3.0
       for j in range(3)]
  m = jnp.maximum(jnp.maximum(t[0], t[1]), t[2])
  e = [jnp.exp(tj - m) for tj in t]
  s = e[0] + e[1] + e[2]
  return 3.0 * ((e[0] / s) * low + (e[1] / s) * high + (e[2] / s) * mlp)


def _deg_inv(degp_ref):
  deg = degp_ref[0, :, 0:1] + degp_ref[1, :, 0:1]
  return 1.0 / jnp.clip(deg, 1.0, None)


def _mid_body(xwh_ref, xwm_ref, lowp_ref, highp_ref, degp_ref,
              al_ref, ah_ref, am_ref, att_ref, lng_ref, lnb_ref,
              wlh2_ref, wm2_ref, hlh_ref, hm_ref):
  dinv = _deg_inv(degp_ref)
  low = jax.nn.relu(dinv * (lowp_ref[0] + lowp_ref[1]))
  xwh = xwh_ref[...]
  high = jax.nn.relu(xwh - dinv * (highp_ref[0] + highp_ref[1]))
  mlp = jax.nn.relu(xwm_ref[...])
  h = _mix(low, high, mlp, al_ref[...], ah_ref[...], am_ref[...], att_ref)
  mean = jnp.mean(h, axis=1, keepdims=True)
  cent = h - mean
  var = jnp.mean(cent * cent, axis=1, keepdims=True)
  h = cent / jnp.sqrt(var + 1e-9) * lng_ref[...] + lnb_ref[...]
  h = jax.nn.relu(h)
  hlh_ref[...] = jnp.dot(h, wlh2_ref[...], preferred_element_type=jnp.float32)
  hm_ref[...] = jnp.dot(h, wm2_ref[...], preferred_element_type=jnp.float32)


def _post_body(hlh_ref, hm_ref, p2_ref, degp_ref,
               al_ref, ah_ref, am_ref, att_ref, out_ref):
  dinv = _deg_inv(degp_ref)
  s2 = p2_ref[0] + p2_ref[1]
  low = jax.nn.relu(dinv * s2[:, 0:64])
  high = jax.nn.relu(hlh_ref[:, 64:128] - dinv * s2[:, 64:128])
  mlp = jax.nn.relu(hm_ref[...])
  out_ref[...] = _mix(low, high, mlp, al_ref[...], ah_ref[...], am_ref[...],
                      att_ref)


def _row_spec(d):
  return pl.BlockSpec((RB, d), lambda i: (i, 0))


def _full_spec(shape):
  return pl.BlockSpec(shape, lambda i: tuple(0 for _ in shape))


def _part_spec(d):
  return pl.BlockSpec((NC, RB, d), lambda i: (0, i, 0))


_deg_spec = pl.BlockSpec((NC, RB, D1), lambda i: (0, i, 0))


_smem_spec = pl.BlockSpec(memory_space=pltpu.SMEM)

_grid = (N // RB,)

_pre_call = pl.pallas_call(
    _pre_body,
    grid=_grid,
    in_specs=[_row_spec(128), _full_spec((128, 128)), _full_spec((128, 128)),
              _full_spec((128, 128))],
    out_specs=[_row_spec(128), _row_spec(128), _row_spec(128)],
    out_shape=[jax.ShapeDtypeStruct((N, 128), jnp.float32)] * 3,
)

_mid_call = pl.pallas_call(
    _mid_body,
    grid=_grid,
    in_specs=[
        _row_spec(128), _row_spec(128),
        _part_spec(D1), _part_spec(D1), _deg_spec,
        _full_spec((1, 128)), _full_spec((1, 128)), _full_spec((1, 128)),
        _smem_spec,
        _full_spec((1, 128)), _full_spec((1, 128)),
        _full_spec((128, 128)), _full_spec((128, 64)),
    ],
    out_specs=[_row_spec(128), _row_spec(64)],
    out_shape=[jax.ShapeDtypeStruct((N, 128), jnp.float32),
               jax.ShapeDtypeStruct((N, 64), jnp.float32)],
)

_post_call = pl.pallas_call(
    _post_body,
    grid=_grid,
    in_specs=[
        _row_spec(128), _row_spec(64),
        _part_spec(D1), _deg_spec,
        _full_spec((1, 64)), _full_spec((1, 64)), _full_spec((1, 64)),
        _smem_spec,
    ],
    out_specs=_row_spec(64),
    out_shape=jax.ShapeDtypeStruct((N, 64), jnp.float32),
)


@jax.jit
def kernel(x, edge_index, W_low1, W_high1, W_mlp1, a_low1, a_high1, a_mlp1,
           att1, ln_g, ln_b, W_low2, W_high2, W_mlp2, a_low2, a_high2,
           a_mlp2, att2):
  row = edge_index[0]
  col = edge_index[1]

  sc1, sc2 = _sc_kernels()
  xwl, xwh, xwm = _pre_call(x, W_low1, W_high1, W_mlp1)
  z128 = jnp.zeros((CH, D1), jnp.float32)
  ones128 = jnp.ones((CH, D1), jnp.float32)
  lowp, highp, degp = sc1(xwl, xwh, row, col, z128, ones128)

  wlh2 = jnp.concatenate([W_low2, W_high2], axis=1)
  hlh, hm = _mid_call(
      xwh, xwm, lowp, highp, degp,
      a_low1.reshape(1, 128), a_high1.reshape(1, 128), a_mlp1.reshape(1, 128),
      att1, ln_g.reshape(1, 128), ln_b.reshape(1, 128), wlh2, W_mlp2)

  (p2,) = sc2(hlh, row, col, z128)
  out = _post_call(
      hlh, hm, p2, degp,
      a_low2.reshape(1, 64), a_high2.reshape(1, 64), a_mlp2.reshape(1, 64),
      att2)
  return out


# speedup vs baseline: 1.1367x; 1.1367x over previous
import functools

import jax
import jax.numpy as jnp
from jax import lax
from jax.experimental import pallas as pl
from jax.experimental.pallas import tpu as pltpu
from jax.experimental.pallas import tpu_sc as plsc

N = 10000
NP = 10240
E = 320000
D1 = 128
NC = 2
NS = 16
NW = NC * NS
EPW = E // NW
CH = 80
NCHUNK = EPW // CH
RPT = NP // NS
RB = 2000


CPS = RPT // CH
VPC = CH // 16


def _zero_stripe(accum, zsrc, sid, sem):
  ds = []
  for j in range(CPS):
    ds.append(pltpu.async_copy(
        zsrc, accum.at[pl.ds(sid * RPT + j * CH, CH)], sem))
  for d in ds:
    d.wait()


def _copy_out(accum, b0, b1, out_hbm, cid, sid, sem):
  bs = (b0, b1)
  ds = []
  for j in range(CPS):
    b = bs[j % 2]
    if j >= 2:
      ds[j - 2].wait()
    base = sid * RPT + j * CH
    pltpu.sync_copy(accum.at[pl.ds(base, CH)], b)
    ds.append(pltpu.async_copy(b, out_hbm.at[cid, pl.ds(base, CH)], sem))
  ds[-2].wait()
  ds[-1].wait()


def _prop_pass(table_hbm, row_hbm, col_hbm, accum, slots, wid):
  def idx_start(k, s):
    cb, rb, _, _, isem = s
    base = wid * EPW + k * CH
    r = pltpu.async_copy(row_hbm.at[pl.ds(base, CH)], rb, isem)
    c = pltpu.async_copy(col_hbm.at[pl.ds(base, CH)], cb, isem)
    return r, c

  def idx_wait(d):
    d[0].wait()
    d[1].wait()

  def gather_start(s):
    cb, _, gb, gsem, _ = s
    return pltpu.async_copy(table_hbm.at[cb], gb, gsem)

  def scatter(s):
    _, rb, gb, _, _ = s
    pltpu.sync_copy(gb, accum.at[rb], add=True)

  def gather_wait(s):
    pltpu.make_async_copy(table_hbm.at[s[0]], s[2], s[3]).wait()

  s0, s1, s2 = slots
  d0 = idx_start(0, s0)
  d1 = idx_start(1, s1)
  d2 = idx_start(2, s2)
  idx_wait(d0)
  gather_start(s0)
  idx_wait(d1)
  gather_start(s1)

  def body(i, c):
    k = 3 * i
    gather_wait(s0)
    scatter(s0)
    idx_start(k + 3, s0)
    idx_wait(d2)
    gather_start(s2)
    gather_wait(s1)
    scatter(s1)
    idx_start(k + 4, s1)
    idx_wait(d0)
    gather_start(s0)
    gather_wait(s2)
    scatter(s2)
    idx_start(k + 5, s2)
    idx_wait(d1)
    gather_start(s1)
    return c

  lax.fori_loop(0, (NCHUNK - 5) // 3, body, 0)

  gather_wait(s0)
  scatter(s0)
  d3 = idx_start(NCHUNK - 2, s0)
  idx_wait(d2)
  gather_start(s2)
  gather_wait(s1)
  scatter(s1)
  d4 = idx_start(NCHUNK - 1, s1)
  idx_wait(d3)
  gather_start(s0)
  gather_wait(s2)
  scatter(s2)
  idx_wait(d4)
  gather_start(s1)
  gather_wait(s0)
  scatter(s0)
  gather_wait(s1)
  scatter(s1)


def _deg_pass(row_hbm, accum, ones_b, rb0, rb1, isem0, isem1, wid):
  def load(k, rb, sem):
    base = wid * EPW + k * CH
    return pltpu.async_copy(row_hbm.at[pl.ds(base, CH)], rb, sem)

  pltpu.sync_copy(row_hbm.at[pl.ds(wid * EPW, CH)], rb0)

  def body(i, c):
    k0 = 2 * i
    l1 = load(k0 + 1, rb1, isem1)
    pltpu.sync_copy(ones_b, accum.at[rb0], add=True)
    l1.wait()
    l0 = load(k0 + 2, rb0, isem0)
    pltpu.sync_copy(ones_b, accum.at[rb1], add=True)
    l0.wait()
    return c

  lax.fori_loop(0, (NCHUNK - 1) // 2, body, 0)
  pltpu.sync_copy(ones_b, accum.at[rb0], add=True)


def _sc1_body(xwl, xwh, row_h, col_h, z128_h, ones_h,
              low_o, high_o, deg_o,
              cb0, rb0, gb0, cb1, rb1, gb1, cb2, rb2, gb2, accum,
              gsem0, gsem1, gsem2, isem0, isem1, isem2):
  cid = lax.axis_index("c")
  sid = lax.axis_index("s")
  wid = sid * NC + cid
  slots = ((cb0, rb0, gb0, gsem0, isem0),
           (cb1, rb1, gb1, gsem1, isem1),
           (cb2, rb2, gb2, gsem2, isem2))

  pltpu.sync_copy(z128_h, gb0)
  _zero_stripe(accum, gb0, sid, gsem0)
  pltpu.sync_copy(ones_h, gb0)
  plsc.subcore_barrier()

  _deg_pass(row_h, accum, gb0, rb0, rb1, isem0, isem1, wid)
  plsc.subcore_barrier()
  _copy_out(accum, gb1, gb2, deg_o, cid, sid, gsem0)
  pltpu.sync_copy(z128_h, gb0)
  _zero_stripe(accum, gb0, sid, gsem1)
  plsc.subcore_barrier()

  _prop_pass(xwl, row_h, col_h, accum, slots, wid)
  plsc.subcore_barrier()
  _copy_out(accum, gb1, gb2, low_o, cid, sid, gsem0)
  pltpu.sync_copy(z128_h, gb0)
  _zero_stripe(accum, gb0, sid, gsem1)
  plsc.subcore_barrier()

  _prop_pass(xwh, row_h, col_h, accum, slots, wid)
  plsc.subcore_barrier()
  _copy_out(accum, gb1, gb2, high_o, cid, sid, gsem0)


def _sc2_body(hlh, row_h, col_h, z128_h, out_o,
              cb0, rb0, gb0, cb1, rb1, gb1, cb2, rb2, gb2, accum,
              gsem0, gsem1, gsem2, isem0, isem1, isem2):
  cid = lax.axis_index("c")
  sid = lax.axis_index("s")
  wid = sid * NC + cid
  slots = ((cb0, rb0, gb0, gsem0, isem0),
           (cb1, rb1, gb1, gsem1, isem1),
           (cb2, rb2, gb2, gsem2, isem2))

  pltpu.sync_copy(z128_h, gb0)
  _zero_stripe(accum, gb0, sid, gsem0)
  plsc.subcore_barrier()

  _prop_pass(hlh, row_h, col_h, accum, slots, wid)
  plsc.subcore_barrier()
  _copy_out(accum, gb1, gb2, out_o, cid, sid, gsem0)


@functools.cache
def _sc_kernels():
  mesh = plsc.VectorSubcoreMesh(core_axis_name="c", subcore_axis_name="s",
                                num_cores=NC, num_subcores=NS)
  sc1 = functools.partial(
      pl.kernel,
      out_type=[
          jax.ShapeDtypeStruct((NC, NP, D1), jnp.float32),
          jax.ShapeDtypeStruct((NC, NP, D1), jnp.float32),
          jax.ShapeDtypeStruct((NC, NP, D1), jnp.float32),
      ],
      mesh=mesh,
      scratch_types=[
          pltpu.VMEM((CH,), jnp.int32),
          pltpu.VMEM((CH,), jnp.int32),
          pltpu.VMEM((CH, D1), jnp.float32),
          pltpu.VMEM((CH,), jnp.int32),
          pltpu.VMEM((CH,), jnp.int32),
          pltpu.VMEM((CH, D1), jnp.float32),
          pltpu.VMEM((CH,), jnp.int32),
          pltpu.VMEM((CH,), jnp.int32),
          pltpu.VMEM((CH, D1), jnp.float32),
          pltpu.VMEM_SHARED((NP, D1), jnp.float32),
          pltpu.SemaphoreType.DMA,
          pltpu.SemaphoreType.DMA,
          pltpu.SemaphoreType.DMA,
          pltpu.SemaphoreType.DMA,
          pltpu.SemaphoreType.DMA,
          pltpu.SemaphoreType.DMA,
      ],
  )(_sc1_body)

  sc2 = functools.partial(
      pl.kernel,
      out_type=[jax.ShapeDtypeStruct((NC, NP, D1), jnp.float32)],
      mesh=mesh,
      scratch_types=[
          pltpu.VMEM((CH,), jnp.int32),
          pltpu.VMEM((CH,), jnp.int32),
          pltpu.VMEM((CH, D1), jnp.float32),
          pltpu.VMEM((CH,), jnp.int32),
          pltpu.VMEM((CH,), jnp.int32),
          pltpu.VMEM((CH, D1), jnp.float32),
          pltpu.VMEM((CH,), jnp.int32),
          pltpu.VMEM((CH,), jnp.int32),
          pltpu.VMEM((CH, D1), jnp.float32),
          pltpu.VMEM_SHARED((NP, D1), jnp.float32),
          pltpu.SemaphoreType.DMA,
          pltpu.SemaphoreType.DMA,
          pltpu.SemaphoreType.DMA,
          pltpu.SemaphoreType.DMA,
          pltpu.SemaphoreType.DMA,
          pltpu.SemaphoreType.DMA,
      ],
  )(_sc2_body)
  return sc1, sc2


def _pre_body(x_ref, wl_ref, wh_ref, wm_ref, ol_ref, oh_ref, om_ref):
  xb = x_ref[...]
  ol_ref[...] = jnp.dot(xb, wl_ref[...], preferred_element_type=jnp.float32)
  oh_ref[...] = jnp.dot(xb, wh_ref[...], preferred_element_type=jnp.float32)
  om_ref[...] = jnp.dot(xb, wm_ref[...], preferred_element_type=jnp.float32)


def _mix(low, high, mlp, al, ah, am, av_ref):
  sl = jnp.sum(low * al, axis=1, keepdims=True)
  sh = jnp.sum(high * ah, axis=1, keepdims=True)
  sm = jnp.sum(mlp * am, axis=1, keepdims=True)
  g0 = jax.nn.sigmoid(sl)
  g1 = jax.nn.sigmoid(sh)
  g2 = jax.nn.sigmoid(sm)
  t = [(g0 * av_ref[0, j] + g1 * av_ref[1, j] + g2 * av_ref[2, j]) / 3.0
       for j in range(3)]
  m = jnp.maximum(jnp.maximum(t[0], t[1]), t[2])
  e = [jnp.exp(tj - m) for tj in t]
  s = e[0] + e[1] + e[2]
  return 3.0 * ((e[0] / s) * low + (e[1] / s) * high + (e[2] / s) * mlp)


def _deg_inv(degp_ref):
  deg = degp_ref[0, :, 0:1] + degp_ref[1, :, 0:1]
  return 1.0 / jnp.clip(deg, 1.0, None)


def _mid_body(xwh_ref, xwm_ref, lowp_ref, highp_ref, degp_ref,
              al_ref, ah_ref, am_ref, att_ref, lng_ref, lnb_ref,
              wlh2_ref, wm2_ref, hlh_ref, hm_ref):
  dinv = _deg_inv(degp_ref)
  low = jax.nn.relu(dinv * (lowp_ref[0] + lowp_ref[1]))
  xwh = xwh_ref[...]
  high = jax.nn.relu(xwh - dinv * (highp_ref[0] + highp_ref[1]))
  mlp = jax.nn.relu(xwm_ref[...])
  h = _mix(low, high, mlp, al_ref[...], ah_ref[...], am_ref[...], att_ref)
  mean = jnp.mean(h, axis=1, keepdims=True)
  cent = h - mean
  var = jnp.mean(cent * cent, axis=1, keepdims=True)
  h = cent / jnp.sqrt(var + 1e-9) * lng_ref[...] + lnb_ref[...]
  h = jax.nn.relu(h)
  hlh_ref[...] = jnp.dot(h, wlh2_ref[...], preferred_element_type=jnp.float32)
  hm_ref[...] = jnp.dot(h, wm2_ref[...], preferred_element_type=jnp.float32)


def _post_body(hlh_ref, hm_ref, p2_ref, degp_ref,
               al_ref, ah_ref, am_ref, att_ref, out_ref):
  dinv = _deg_inv(degp_ref)
  s2 = p2_ref[0] + p2_ref[1]
  low = jax.nn.relu(dinv * s2[:, 0:64])
  high = jax.nn.relu(hlh_ref[:, 64:128] - dinv * s2[:, 64:128])
  mlp = jax.nn.relu(hm_ref[...])
  out_ref[...] = _mix(low, high, mlp, al_ref[...], ah_ref[...], am_ref[...],
                      att_ref)


def _row_spec(d):
  return pl.BlockSpec((RB, d), lambda i: (i, 0))


def _full_spec(shape):
  return pl.BlockSpec(shape, lambda i: tuple(0 for _ in shape))


def _part_spec(d):
  return pl.BlockSpec((NC, RB, d), lambda i: (0, i, 0))


_deg_spec = pl.BlockSpec((NC, RB, D1), lambda i: (0, i, 0))


_smem_spec = pl.BlockSpec(memory_space=pltpu.SMEM)

_grid = (N // RB,)

_pre_call = pl.pallas_call(
    _pre_body,
    grid=_grid,
    in_specs=[_row_spec(128), _full_spec((128, 128)), _full_spec((128, 128)),
              _full_spec((128, 128))],
    out_specs=[_row_spec(128), _row_spec(128), _row_spec(128)],
    out_shape=[jax.ShapeDtypeStruct((N, 128), jnp.float32)] * 3,
)

_mid_call = pl.pallas_call(
    _mid_body,
    grid=_grid,
    in_specs=[
        _row_spec(128), _row_spec(128),
        _part_spec(D1), _part_spec(D1), _deg_spec,
        _full_spec((1, 128)), _full_spec((1, 128)), _full_spec((1, 128)),
        _smem_spec,
        _full_spec((1, 128)), _full_spec((1, 128)),
        _full_spec((128, 128)), _full_spec((128, 64)),
    ],
    out_specs=[_row_spec(128), _row_spec(64)],
    out_shape=[jax.ShapeDtypeStruct((N, 128), jnp.float32),
               jax.ShapeDtypeStruct((N, 64), jnp.float32)],
)

_post_call = pl.pallas_call(
    _post_body,
    grid=_grid,
    in_specs=[
        _row_spec(128), _row_spec(64),
        _part_spec(D1), _deg_spec,
        _full_spec((1, 64)), _full_spec((1, 64)), _full_spec((1, 64)),
        _smem_spec,
    ],
    out_specs=_row_spec(64),
    out_shape=jax.ShapeDtypeStruct((N, 64), jnp.float32),
)


@jax.jit
def kernel(x, edge_index, W_low1, W_high1, W_mlp1, a_low1, a_high1, a_mlp1,
           att1, ln_g, ln_b, W_low2, W_high2, W_mlp2, a_low2, a_high2,
           a_mlp2, att2):
  row = edge_index[0]
  col = edge_index[1]

  sc1, sc2 = _sc_kernels()
  z128 = jnp.zeros((CH, D1), jnp.float32)
  ones128 = jnp.ones((CH, D1), jnp.float32)
  xwl, xwh, xwm = _pre_call(x, W_low1, W_high1, W_mlp1)
  lowp, highp, degp = sc1(xwl, xwh, row, col, z128, ones128)

  wlh2 = jnp.concatenate([W_low2, W_high2], axis=1)
  hlh, hm = _mid_call(
      xwh, xwm, lowp, highp, degp,
      a_low1.reshape(1, 128), a_high1.reshape(1, 128), a_mlp1.reshape(1, 128),
      att1, ln_g.reshape(1, 128), ln_b.reshape(1, 128), wlh2, W_mlp2)

  (p2,) = sc2(hlh, row, col, z128)
  out = _post_call(
      hlh, hm, p2, degp,
      a_low2.reshape(1, 64), a_high2.reshape(1, 64), a_mlp2.reshape(1, 64),
      att2)
  return out
